# baseline (speedup 1.0000x reference)
import functools

import jax
import jax.numpy as jnp
from jax.experimental import pallas as pl
from jax.experimental.pallas import tpu as pltpu

D_MODEL = 1024
D_FF = 4096
NUM_EXPERTS = 16
TOP_N = 2
LORA_R = 8
LORA_SCALE = 2.0
THRESHOLD = 0.2
MIN_CAP = 8
EPS = 1e-9
N = 2048
CAP = 160
BN = 512

_P1 = jax.random.uniform(jax.random.key(42), (TOP_N, 1, N))[1]


def _route_kernel(x_ref, lnw_ref, wg_ref, p1_ref,
                  ln_ref, logits_ref, energy_ref, aux_ref,
                  slot0_ref, slot1_ref, g0_ref, g1_ref):
    xv = x_ref[0]
    var = jnp.mean(jnp.square(xv), axis=-1, keepdims=True)
    ln = lnw_ref[0][None, :] * xv * jax.lax.rsqrt(var + 1e-6)
    ln_ref[...] = ln.astype(jnp.bfloat16)

    logits = jnp.dot(ln, wg_ref[...], preferred_element_type=jnp.float32)
    logits_ref[...] = logits

    m = jnp.max(logits, axis=-1, keepdims=True)
    unnorm = jnp.exp(logits - m)
    ssum = jnp.sum(unnorm, axis=-1, keepdims=True)
    lse = m + jnp.log(ssum)
    energy_ref[...] = -lse.reshape(1, N)
    raw = unnorm / ssum

    eids = jax.lax.broadcasted_iota(jnp.int32, (N, NUM_EXPERTS), 1)
    v0 = jnp.max(raw, axis=-1, keepdims=True)
    e0 = jnp.min(jnp.where(raw == v0, eids, NUM_EXPERTS),
                 axis=-1, keepdims=True)
    masked = jnp.where(eids == e0, -jnp.inf, raw)
    v1 = jnp.max(masked, axis=-1, keepdims=True)
    e1 = jnp.min(jnp.where(masked == v1, eids, NUM_EXPERTS),
                 axis=-1, keepdims=True)

    denom = jnp.maximum(v0 + v1, EPS)
    g0 = v0 / denom
    g1 = v1 / denom

    should1 = p1_ref[0][:, None] < (g1 / THRESHOLD)

    oh0 = (eids == e0).astype(jnp.float32)
    oh1 = (eids == e1).astype(jnp.float32)
    mask1 = oh1 * should1.astype(jnp.float32)

    rows = jax.lax.broadcasted_iota(jnp.int32, (N, N), 0)
    cols = jax.lax.broadcasted_iota(jnp.int32, (N, N), 1)
    ltri = (cols < rows).astype(jnp.float32)
    cum0 = jnp.dot(ltri, oh0, preferred_element_type=jnp.float32)
    cum1 = jnp.dot(ltri, mask1, preferred_element_type=jnp.float32)

    pie0 = jnp.sum(cum0 * oh0, axis=-1, keepdims=True)
    keep0 = pie0 < float(CAP)
    prev = jnp.sum(oh0 * keep0.astype(jnp.float32), axis=0, keepdims=True)
    pie1 = jnp.sum((cum1 + prev) * mask1, axis=-1, keepdims=True)
    keep1 = should1 & (pie1 < float(CAP))

    pos0 = pie0.astype(jnp.int32)
    pos1 = pie1.astype(jnp.int32)
    slot0 = jnp.where(keep0, e0 * CAP + pos0, -1)
    slot1 = jnp.where(keep1, e1 * CAP + pos1, -1)
    slot0_ref[...] = slot0.reshape(1, N)
    slot1_ref[...] = slot1.reshape(1, N)
    g0_ref[...] = (g0 * keep0.astype(jnp.float32)).reshape(1, N)
    g1_ref[...] = (g1 * keep1.astype(jnp.float32)).reshape(1, N)

    density_1 = jnp.mean(oh0, axis=0)
    density_proxy = jnp.mean(raw, axis=0)
    balance = jnp.mean(density_proxy * density_1) * float(NUM_EXPERTS ** 2)
    z_loss = jnp.mean(jnp.square(lse))
    aux = 0.01 * balance + 0.01 * z_loss
    aux_ref[...] = jnp.broadcast_to(aux, (1, 1))


def _expert_kernel(ln_ref, slot0_ref, slot1_ref,
                   wi_ref, wo_ref, wia_ref, wib_ref, woa_ref, wob_ref,
                   out_ref):
    e = pl.program_id(0)
    base = e * CAP
    cids = base + jax.lax.broadcasted_iota(jnp.int32, (CAP, N), 0)
    d0 = (cids == slot0_ref[...]).astype(jnp.bfloat16)
    d1 = (cids == slot1_ref[...]).astype(jnp.bfloat16)

    xe = jnp.dot(d0 + d1, ln_ref[...],
                 preferred_element_type=jnp.float32)

    ai = wia_ref[0]
    bi = wib_ref[0]
    h = jnp.dot(xe, wi_ref[...], preferred_element_type=jnp.float32)
    h += LORA_SCALE * jnp.dot(
        jnp.dot(xe, ai, preferred_element_type=jnp.float32), bi,
        preferred_element_type=jnp.float32)
    h = jnp.maximum(h, 0.0)

    ao = woa_ref[0]
    bo = wob_ref[0]
    ye = jnp.dot(h, wo_ref[...], preferred_element_type=jnp.float32)
    ye += LORA_SCALE * jnp.dot(
        jnp.dot(h, ao, preferred_element_type=jnp.float32), bo,
        preferred_element_type=jnp.float32)

    out_ref[0] = ye


def _combine_kernel(x_ref, y_ref, slot0_ref, slot1_ref, g0_ref, g1_ref,
                    out_ref):
    sid = jax.lax.broadcasted_iota(jnp.int32, (NUM_EXPERTS * CAP, BN), 0)
    c0 = jnp.where(sid == slot0_ref[...], g0_ref[...], 0.0)
    c1 = jnp.where(sid == slot1_ref[...], g1_ref[...], 0.0)
    comb = c0 + c1
    out_ref[...] = x_ref[...] + jax.lax.dot_general(
        comb, y_ref[...], (((0,), (0,)), ((), ())),
        preferred_element_type=jnp.float32)


@jax.jit
def kernel(x, ln_w, Wi, Wo, Wg, lora_wi_A, lora_wi_B, lora_wo_A, lora_wo_B):
    ln, logits, energy, aux, slot0, slot1, g0, g1 = pl.pallas_call(
        _route_kernel,
        out_shape=[
            jax.ShapeDtypeStruct((N, D_MODEL), jnp.bfloat16),
            jax.ShapeDtypeStruct((N, NUM_EXPERTS), jnp.float32),
            jax.ShapeDtypeStruct((1, N), jnp.float32),
            jax.ShapeDtypeStruct((1, 1), jnp.float32),
            jax.ShapeDtypeStruct((1, N), jnp.int32),
            jax.ShapeDtypeStruct((1, N), jnp.int32),
            jax.ShapeDtypeStruct((1, N), jnp.float32),
            jax.ShapeDtypeStruct((1, N), jnp.float32),
        ],
    )(x, ln_w.reshape(1, D_MODEL), Wg, _P1)

    const2 = lambda e: (0, 0)
    ye_all = pl.pallas_call(
        _expert_kernel,
        grid=(NUM_EXPERTS,),
        in_specs=[
            pl.BlockSpec((N, D_MODEL), const2),
            pl.BlockSpec((1, N), const2),
            pl.BlockSpec((1, N), const2),
            pl.BlockSpec((D_MODEL, D_FF), const2),
            pl.BlockSpec((D_FF, D_MODEL), const2),
            pl.BlockSpec((1, D_MODEL, LORA_R), lambda e: (e, 0, 0)),
            pl.BlockSpec((1, LORA_R, D_FF), lambda e: (e, 0, 0)),
            pl.BlockSpec((1, D_FF, LORA_R), lambda e: (e, 0, 0)),
            pl.BlockSpec((1, LORA_R, D_MODEL), lambda e: (e, 0, 0)),
        ],
        out_specs=pl.BlockSpec((1, CAP, D_MODEL), lambda e: (e, 0, 0)),
        out_shape=jax.ShapeDtypeStruct((NUM_EXPERTS, CAP, D_MODEL),
                                       jnp.float32),
    )(ln, slot0, slot1, Wi, Wo,
      lora_wi_A, lora_wi_B, lora_wo_A, lora_wo_B)

    out = pl.pallas_call(
        _combine_kernel,
        grid=(N // BN,),
        in_specs=[
            pl.BlockSpec((BN, D_MODEL), lambda i: (i, 0)),
            pl.BlockSpec((NUM_EXPERTS * CAP, D_MODEL), const2),
            pl.BlockSpec((1, BN), lambda i: (0, i)),
            pl.BlockSpec((1, BN), lambda i: (0, i)),
            pl.BlockSpec((1, BN), lambda i: (0, i)),
            pl.BlockSpec((1, BN), lambda i: (0, i)),
        ],
        out_specs=pl.BlockSpec((BN, D_MODEL), lambda i: (i, 0)),
        out_shape=jax.ShapeDtypeStruct((N, D_MODEL), jnp.float32),
    )(x.reshape(N, D_MODEL), ye_all.reshape(NUM_EXPERTS * CAP, D_MODEL),
      slot0, slot1, g0, g1)

    return (out.reshape(1, N, D_MODEL), aux[0, 0],
            logits.reshape(1, N, NUM_EXPERTS), energy)


# speedup vs baseline: 1.1538x; 1.1538x over previous
import functools

import jax
import jax.numpy as jnp
from jax.experimental import pallas as pl
from jax.experimental.pallas import tpu as pltpu

D_MODEL = 1024
D_FF = 4096
NUM_EXPERTS = 16
TOP_N = 2
LORA_R = 8
LORA_SCALE = 2.0
THRESHOLD = 0.2
MIN_CAP = 8
EPS = 1e-9
N = 2048
CAP = 160
BN = 512

_P1 = jax.random.uniform(jax.random.key(42), (TOP_N, 1, N))[1]


def _route_kernel(x_ref, lnw_ref, wg_ref, p1_ref,
                  ln_ref, logits_ref, energy_ref, aux_ref,
                  slot0_ref, slot1_ref, g0_ref, g1_ref):
    xv = x_ref[0]
    var = jnp.mean(jnp.square(xv), axis=-1, keepdims=True)
    ln = lnw_ref[0][None, :] * xv * jax.lax.rsqrt(var + 1e-6)
    ln_ref[...] = ln.astype(jnp.bfloat16)

    logits = jnp.dot(ln, wg_ref[...], preferred_element_type=jnp.float32)
    logits_ref[...] = logits

    m = jnp.max(logits, axis=-1, keepdims=True)
    unnorm = jnp.exp(logits - m)
    ssum = jnp.sum(unnorm, axis=-1, keepdims=True)
    lse = m + jnp.log(ssum)
    energy_ref[...] = -lse.reshape(1, N)
    raw = unnorm / ssum

    eids = jax.lax.broadcasted_iota(jnp.int32, (N, NUM_EXPERTS), 1)
    v0 = jnp.max(raw, axis=-1, keepdims=True)
    e0 = jnp.min(jnp.where(raw == v0, eids, NUM_EXPERTS),
                 axis=-1, keepdims=True)
    masked = jnp.where(eids == e0, -jnp.inf, raw)
    v1 = jnp.max(masked, axis=-1, keepdims=True)
    e1 = jnp.min(jnp.where(masked == v1, eids, NUM_EXPERTS),
                 axis=-1, keepdims=True)

    denom = jnp.maximum(v0 + v1, EPS)
    g0 = v0 / denom
    g1 = v1 / denom

    should1 = p1_ref[0][:, None] < (g1 / THRESHOLD)

    oh0 = (eids == e0).astype(jnp.float32)
    oh1 = (eids == e1).astype(jnp.float32)
    mask1 = oh1 * should1.astype(jnp.float32)

    rows = jax.lax.broadcasted_iota(jnp.int32, (N, N), 0)
    cols = jax.lax.broadcasted_iota(jnp.int32, (N, N), 1)
    ltri = (cols < rows).astype(jnp.float32)
    cum0 = jnp.dot(ltri, oh0, preferred_element_type=jnp.float32)
    cum1 = jnp.dot(ltri, mask1, preferred_element_type=jnp.float32)

    pie0 = jnp.sum(cum0 * oh0, axis=-1, keepdims=True)
    keep0 = pie0 < float(CAP)
    prev = jnp.sum(oh0 * keep0.astype(jnp.float32), axis=0, keepdims=True)
    pie1 = jnp.sum((cum1 + prev) * mask1, axis=-1, keepdims=True)
    keep1 = should1 & (pie1 < float(CAP))

    pos0 = pie0.astype(jnp.int32)
    pos1 = pie1.astype(jnp.int32)
    slot0 = jnp.where(keep0, e0 * CAP + pos0, -1)
    slot1 = jnp.where(keep1, e1 * CAP + pos1, -1)
    slot0_ref[...] = slot0.reshape(1, N)
    slot1_ref[...] = slot1.reshape(1, N)
    g0_ref[...] = (g0 * keep0.astype(jnp.float32)).reshape(1, N)
    g1_ref[...] = (g1 * keep1.astype(jnp.float32)).reshape(1, N)

    density_1 = jnp.mean(oh0, axis=0)
    density_proxy = jnp.mean(raw, axis=0)
    balance = jnp.mean(density_proxy * density_1) * float(NUM_EXPERTS ** 2)
    z_loss = jnp.mean(jnp.square(lse))
    aux = 0.01 * balance + 0.01 * z_loss
    aux_ref[...] = jnp.broadcast_to(aux, (1, 1))


def _expert_kernel(ln_ref, slot0_ref, slot1_ref,
                   wi_ref, wo_ref, wiat_ref, wib_ref, woat_ref, wob_ref,
                   out_ref):
    e = pl.program_id(0)
    base = e * CAP
    cids = base + jax.lax.broadcasted_iota(jnp.int32, (CAP, N), 0)
    d0 = (cids == slot0_ref[...]).astype(jnp.bfloat16)
    d1 = (cids == slot1_ref[...]).astype(jnp.bfloat16)

    xe = jnp.dot(d0 + d1, ln_ref[...],
                 preferred_element_type=jnp.float32)

    ait = wiat_ref[0]
    bi = wib_ref[0]
    h = jnp.dot(xe, wi_ref[...], preferred_element_type=jnp.float32)
    h += LORA_SCALE * jnp.dot(
        jax.lax.dot_general(xe, ait, (((1,), (1,)), ((), ())),
                            preferred_element_type=jnp.float32), bi,
        preferred_element_type=jnp.float32)
    h = jnp.maximum(h, 0.0)

    aot = woat_ref[0]
    bo = wob_ref[0]
    ye = jnp.dot(h, wo_ref[...], preferred_element_type=jnp.float32)
    ye += LORA_SCALE * jnp.dot(
        jax.lax.dot_general(h, aot, (((1,), (1,)), ((), ())),
                            preferred_element_type=jnp.float32), bo,
        preferred_element_type=jnp.float32)

    out_ref[0] = ye


def _combine_kernel(x_ref, y_ref, slot0_ref, slot1_ref, g0_ref, g1_ref,
                    out_ref):
    sid = jax.lax.broadcasted_iota(jnp.int32, (NUM_EXPERTS * CAP, BN), 0)
    c0 = jnp.where(sid == slot0_ref[...], g0_ref[...], 0.0)
    c1 = jnp.where(sid == slot1_ref[...], g1_ref[...], 0.0)
    comb = c0 + c1
    out_ref[...] = x_ref[...] + jax.lax.dot_general(
        comb, y_ref[...], (((0,), (0,)), ((), ())),
        preferred_element_type=jnp.float32)


@jax.jit
def kernel(x, ln_w, Wi, Wo, Wg, lora_wi_A, lora_wi_B, lora_wo_A, lora_wo_B):
    ln, logits, energy, aux, slot0, slot1, g0, g1 = pl.pallas_call(
        _route_kernel,
        out_shape=[
            jax.ShapeDtypeStruct((N, D_MODEL), jnp.bfloat16),
            jax.ShapeDtypeStruct((N, NUM_EXPERTS), jnp.float32),
            jax.ShapeDtypeStruct((1, N), jnp.float32),
            jax.ShapeDtypeStruct((1, 1), jnp.float32),
            jax.ShapeDtypeStruct((1, N), jnp.int32),
            jax.ShapeDtypeStruct((1, N), jnp.int32),
            jax.ShapeDtypeStruct((1, N), jnp.float32),
            jax.ShapeDtypeStruct((1, N), jnp.float32),
        ],
    )(x, ln_w.reshape(1, D_MODEL), Wg, _P1)

    wiat = jnp.transpose(lora_wi_A, (0, 2, 1))
    woat = jnp.transpose(lora_wo_A, (0, 2, 1))
    const2 = lambda e: (0, 0)
    ye_all = pl.pallas_call(
        _expert_kernel,
        grid=(NUM_EXPERTS,),
        in_specs=[
            pl.BlockSpec((N, D_MODEL), const2),
            pl.BlockSpec((1, N), const2),
            pl.BlockSpec((1, N), const2),
            pl.BlockSpec((D_MODEL, D_FF), const2),
            pl.BlockSpec((D_FF, D_MODEL), const2),
            pl.BlockSpec((1, LORA_R, D_MODEL), lambda e: (e, 0, 0)),
            pl.BlockSpec((1, LORA_R, D_FF), lambda e: (e, 0, 0)),
            pl.BlockSpec((1, LORA_R, D_FF), lambda e: (e, 0, 0)),
            pl.BlockSpec((1, LORA_R, D_MODEL), lambda e: (e, 0, 0)),
        ],
        out_specs=pl.BlockSpec((1, CAP, D_MODEL), lambda e: (e, 0, 0)),
        out_shape=jax.ShapeDtypeStruct((NUM_EXPERTS, CAP, D_MODEL),
                                       jnp.float32),
    )(ln, slot0, slot1, Wi, Wo,
      wiat, lora_wi_B, woat, lora_wo_B)

    out = pl.pallas_call(
        _combine_kernel,
        grid=(N // BN,),
        in_specs=[
            pl.BlockSpec((BN, D_MODEL), lambda i: (i, 0)),
            pl.BlockSpec((NUM_EXPERTS * CAP, D_MODEL), const2),
            pl.BlockSpec((1, BN), lambda i: (0, i)),
            pl.BlockSpec((1, BN), lambda i: (0, i)),
            pl.BlockSpec((1, BN), lambda i: (0, i)),
            pl.BlockSpec((1, BN), lambda i: (0, i)),
        ],
        out_specs=pl.BlockSpec((BN, D_MODEL), lambda i: (i, 0)),
        out_shape=jax.ShapeDtypeStruct((N, D_MODEL), jnp.float32),
    )(x.reshape(N, D_MODEL), ye_all.reshape(NUM_EXPERTS * CAP, D_MODEL),
      slot0, slot1, g0, g1)

    return (out.reshape(1, N, D_MODEL), aux[0, 0],
            logits.reshape(1, N, NUM_EXPERTS), energy)


# speedup vs baseline: 1.1959x; 1.0365x over previous
import functools

import jax
import jax.numpy as jnp
from jax.experimental import pallas as pl
from jax.experimental.pallas import tpu as pltpu

D_MODEL = 1024
D_FF = 4096
NUM_EXPERTS = 16
TOP_N = 2
LORA_R = 8
LORA_SCALE = 2.0
THRESHOLD = 0.2
MIN_CAP = 8
EPS = 1e-9
N = 2048
CAP = 160
BN = 256

_P1 = jax.random.uniform(jax.random.key(42), (TOP_N, 1, N))[1]


def _route_kernel(x_ref, lnw_ref, wg_ref, p1_ref,
                  ln_ref, logits_ref, energy_ref, aux_ref,
                  slot0_ref, slot1_ref, g0_ref, g1_ref):
    xv = x_ref[0]
    var = jnp.mean(jnp.square(xv), axis=-1, keepdims=True)
    ln = lnw_ref[0][None, :] * xv * jax.lax.rsqrt(var + 1e-6)
    ln_ref[...] = ln.astype(jnp.bfloat16)

    logits = jnp.dot(ln, wg_ref[...], preferred_element_type=jnp.float32)
    logits_ref[...] = logits

    m = jnp.max(logits, axis=-1, keepdims=True)
    unnorm = jnp.exp(logits - m)
    ssum = jnp.sum(unnorm, axis=-1, keepdims=True)
    lse = m + jnp.log(ssum)
    energy_ref[...] = -lse.reshape(1, N)
    raw = unnorm / ssum

    eids = jax.lax.broadcasted_iota(jnp.int32, (N, NUM_EXPERTS), 1)
    v0 = jnp.max(raw, axis=-1, keepdims=True)
    e0 = jnp.min(jnp.where(raw == v0, eids, NUM_EXPERTS),
                 axis=-1, keepdims=True)
    masked = jnp.where(eids == e0, -jnp.inf, raw)
    v1 = jnp.max(masked, axis=-1, keepdims=True)
    e1 = jnp.min(jnp.where(masked == v1, eids, NUM_EXPERTS),
                 axis=-1, keepdims=True)

    denom = jnp.maximum(v0 + v1, EPS)
    g0 = v0 / denom
    g1 = v1 / denom

    should1 = p1_ref[0][:, None] < (g1 / THRESHOLD)

    oh0 = (eids == e0).astype(jnp.float32)
    oh1 = (eids == e1).astype(jnp.float32)
    mask1 = oh1 * should1.astype(jnp.float32)

    rows = jax.lax.broadcasted_iota(jnp.int32, (N, N), 0)
    cols = jax.lax.broadcasted_iota(jnp.int32, (N, N), 1)
    ltri = (cols < rows).astype(jnp.float32)
    cum0 = jnp.dot(ltri, oh0, preferred_element_type=jnp.float32)
    cum1 = jnp.dot(ltri, mask1, preferred_element_type=jnp.float32)

    pie0 = jnp.sum(cum0 * oh0, axis=-1, keepdims=True)
    keep0 = pie0 < float(CAP)
    prev = jnp.sum(oh0 * keep0.astype(jnp.float32), axis=0, keepdims=True)
    pie1 = jnp.sum((cum1 + prev) * mask1, axis=-1, keepdims=True)
    keep1 = should1 & (pie1 < float(CAP))

    pos0 = pie0.astype(jnp.int32)
    pos1 = pie1.astype(jnp.int32)
    slot0 = jnp.where(keep0, e0 * CAP + pos0, -1)
    slot1 = jnp.where(keep1, e1 * CAP + pos1, -1)
    slot0_ref[...] = slot0.reshape(1, N)
    slot1_ref[...] = slot1.reshape(1, N)
    g0_ref[...] = (g0 * keep0.astype(jnp.float32)).reshape(1, N)
    g1_ref[...] = (g1 * keep1.astype(jnp.float32)).reshape(1, N)

    density_1 = jnp.mean(oh0, axis=0)
    density_proxy = jnp.mean(raw, axis=0)
    balance = jnp.mean(density_proxy * density_1) * float(NUM_EXPERTS ** 2)
    z_loss = jnp.mean(jnp.square(lse))
    aux = 0.01 * balance + 0.01 * z_loss
    aux_ref[...] = jnp.broadcast_to(aux, (1, 1))


def _moe_kernel(ln_ref, slot0_ref, slot1_ref, g0_ref, g1_ref, x_ref,
                wi_ref, wo_ref, wiat_ref, wib_ref, woat_ref, wob_ref,
                out_ref, ye_ref):
    g = pl.program_id(0)

    @pl.when(g < NUM_EXPERTS)
    def _expert():
        _expert_step(g, ln_ref, slot0_ref, slot1_ref,
                     wi_ref, wo_ref, wiat_ref, wib_ref, woat_ref, wob_ref,
                     ye_ref)

    @pl.when(g >= NUM_EXPERTS)
    def _combine():
        i = g - NUM_EXPERTS
        s0 = slot0_ref[:, pl.ds(i * BN, BN)]
        s1 = slot1_ref[:, pl.ds(i * BN, BN)]
        sid = jax.lax.broadcasted_iota(jnp.int32, (NUM_EXPERTS * CAP, BN), 0)
        c0 = jnp.where(sid == s0, g0_ref[:, pl.ds(i * BN, BN)], 0.0)
        c1 = jnp.where(sid == s1, g1_ref[:, pl.ds(i * BN, BN)], 0.0)
        out_ref[...] = x_ref[...] + jax.lax.dot_general(
            c0 + c1, ye_ref[...], (((0,), (0,)), ((), ())),
            preferred_element_type=jnp.float32)


def _expert_step(e, ln_ref, slot0_ref, slot1_ref,
                 wi_ref, wo_ref, wiat_ref, wib_ref, woat_ref, wob_ref,
                 ye_ref):
    base = e * CAP
    cids = base + jax.lax.broadcasted_iota(jnp.int32, (CAP, N), 0)
    d0 = (cids == slot0_ref[...]).astype(jnp.bfloat16)
    d1 = (cids == slot1_ref[...]).astype(jnp.bfloat16)

    xe = jnp.dot(d0 + d1, ln_ref[...],
                 preferred_element_type=jnp.float32)

    ait = wiat_ref[0]
    bi = wib_ref[0]
    h = jnp.dot(xe, wi_ref[...], preferred_element_type=jnp.float32)
    h += LORA_SCALE * jnp.dot(
        jax.lax.dot_general(xe, ait, (((1,), (1,)), ((), ())),
                            preferred_element_type=jnp.float32), bi,
        preferred_element_type=jnp.float32)
    h = jnp.maximum(h, 0.0)

    aot = woat_ref[0]
    bo = wob_ref[0]
    ye = jnp.dot(h, wo_ref[...], preferred_element_type=jnp.float32)
    ye += LORA_SCALE * jnp.dot(
        jax.lax.dot_general(h, aot, (((1,), (1,)), ((), ())),
                            preferred_element_type=jnp.float32), bo,
        preferred_element_type=jnp.float32)

    ye_ref[pl.ds(base, CAP), :] = ye


@jax.jit
def kernel(x, ln_w, Wi, Wo, Wg, lora_wi_A, lora_wi_B, lora_wo_A, lora_wo_B):
    ln, logits, energy, aux, slot0, slot1, g0, g1 = pl.pallas_call(
        _route_kernel,
        out_shape=[
            jax.ShapeDtypeStruct((N, D_MODEL), jnp.bfloat16),
            jax.ShapeDtypeStruct((N, NUM_EXPERTS), jnp.float32),
            jax.ShapeDtypeStruct((1, N), jnp.float32),
            jax.ShapeDtypeStruct((1, 1), jnp.float32),
            jax.ShapeDtypeStruct((1, N), jnp.int32),
            jax.ShapeDtypeStruct((1, N), jnp.int32),
            jax.ShapeDtypeStruct((1, N), jnp.float32),
            jax.ShapeDtypeStruct((1, N), jnp.float32),
        ],
    )(x, ln_w.reshape(1, D_MODEL), Wg, _P1)

    wiat = jnp.transpose(lora_wi_A, (0, 2, 1))
    woat = jnp.transpose(lora_wo_A, (0, 2, 1))
    const2 = lambda g: (0, 0)
    xmap = lambda g: (jnp.maximum(g - NUM_EXPERTS, 0), 0)
    emap = lambda g: (jnp.minimum(g, NUM_EXPERTS - 1), 0, 0)
    out = pl.pallas_call(
        _moe_kernel,
        grid=(NUM_EXPERTS + N // BN,),
        in_specs=[
            pl.BlockSpec((N, D_MODEL), const2),
            pl.BlockSpec((1, N), const2),
            pl.BlockSpec((1, N), const2),
            pl.BlockSpec((1, N), const2),
            pl.BlockSpec((1, N), const2),
            pl.BlockSpec((BN, D_MODEL), xmap),
            pl.BlockSpec((D_MODEL, D_FF), const2),
            pl.BlockSpec((D_FF, D_MODEL), const2),
            pl.BlockSpec((1, LORA_R, D_MODEL), emap),
            pl.BlockSpec((1, LORA_R, D_FF), emap),
            pl.BlockSpec((1, LORA_R, D_FF), emap),
            pl.BlockSpec((1, LORA_R, D_MODEL), emap),
        ],
        out_specs=pl.BlockSpec((BN, D_MODEL), xmap),
        out_shape=jax.ShapeDtypeStruct((N, D_MODEL), jnp.float32),
        scratch_shapes=[
            pltpu.VMEM((NUM_EXPERTS * CAP, D_MODEL), jnp.float32)],
        compiler_params=pltpu.CompilerParams(
            dimension_semantics=("arbitrary",)),
    )(ln, slot0, slot1, g0, g1, x.reshape(N, D_MODEL), Wi, Wo,
      wiat, lora_wi_B, woat, lora_wo_B)

    return (out.reshape(1, N, D_MODEL), aux[0, 0],
            logits.reshape(1, N, NUM_EXPERTS), energy)
